# baseline (speedup 1.0000x reference)
import functools
import math

import jax
import jax.numpy as jnp
from jax import lax
from jax.experimental import pallas as pl
from jax.experimental.pallas import tpu as pltpu
from jax.experimental.pallas import tpu_sc as plsc

EMBED_DIM = 64
SEQ_LEN = 200
NUM_CORES = 2
NUM_SUBCORES = 16
LANES = 16
CHUNK = 400


def _make_sinusoidal_pe(seq_len, dim):
    position = jnp.arange(0, seq_len, dtype=jnp.float32)[:, None]
    div_term = jnp.exp(
        jnp.arange(0, dim, 2, dtype=jnp.float32) * -(math.log(10000.0) / dim)
    )
    pe = jnp.zeros((seq_len, dim), dtype=jnp.float32)
    pe = pe.at[:, 0::2].set(jnp.sin(position * div_term))
    pe = pe.at[:, 1::2].set(jnp.cos(position * div_term))
    return pe


def _sc_embed(ids_flat, table, pe_pair, *, dim, chunk, num_cores,
              num_subcores):
    num_workers = num_cores * num_subcores
    b = ids_flat.shape[0]
    b_per_w = b // num_workers
    n_chunks = b_per_w // chunk
    half = chunk // 2
    mesh = plsc.VectorSubcoreMesh(
        core_axis_name="c", subcore_axis_name="s",
        num_cores=num_cores, num_subcores=num_subcores,
    )

    @functools.partial(
        pl.kernel,
        out_type=jax.ShapeDtypeStruct((b // 2, 2 * dim), jnp.float32),
        mesh=mesh,
        scratch_types=[
            pltpu.VMEM((2, chunk), jnp.int32),
            pltpu.VMEM((chunk, dim), jnp.float32),
            pltpu.VMEM((chunk, dim), jnp.float32),
            pltpu.VMEM((half, 2 * dim), jnp.float32),
            pltpu.VMEM((half, 2 * dim), jnp.float32),
            pltpu.VMEM((half, 2 * dim), jnp.float32),
            pltpu.SemaphoreType.DMA,
            pltpu.SemaphoreType.DMA,
            pltpu.SemaphoreType.DMA,
            pltpu.SemaphoreType.DMA,
            pltpu.SemaphoreType.DMA,
            pltpu.SemaphoreType.DMA,
        ],
        compiler_params=pltpu.CompilerParams(use_tc_tiling_on_sc=False),
    )
    def run(ids_hbm, table_hbm, pe_hbm, out_hbm,
            idx_v, rows0_v, rows1_v, stg0_v, stg1_v, pe_v,
            isem0, isem1, gsem0, gsem1, osem0, osem1):
        rows = (rows0_v, rows1_v)
        stg = (stg0_v, stg1_v)
        isem = (isem0, isem1)
        gsem = (gsem0, gsem1)
        osem = (osem0, osem1)
        wid = lax.axis_index("s") * num_cores + lax.axis_index("c")
        base = wid * b_per_w
        pltpu.sync_copy(pe_hbm, pe_v)

        def idx_copy(bi, g):
            row0 = base + g * chunk
            return pltpu.make_async_copy(
                ids_hbm.at[pl.ds(row0, chunk)], idx_v.at[bi], isem[bi])

        def gather(bi, g):
            return pltpu.make_async_copy(
                table_hbm.at[idx_v.at[bi]], rows[bi], gsem[bi])

        def store(bi, g):
            p0 = (base + g * chunk) // 2
            return pltpu.make_async_copy(
                stg[bi], out_hbm.at[pl.ds(p0, half)], osem[bi])

        idx_copy(0, 0).start()
        idx_copy(1, 1).start()
        idx_copy(0, 0).wait()
        gather(0, 0).start()

        @pl.loop(0, n_chunks, step=2)
        def _chunk_loop(g0):
            for bi in range(2):
                g = g0 + bi
                oth = 1 - bi

                @pl.when(g + 1 < n_chunks)
                def _launch_next_gather():
                    idx_copy(oth, g + 1).wait()
                    gather(oth, g + 1).start()

                gather(bi, g).wait()

                @pl.when(g + 2 < n_chunks)
                def _prefetch_idx():
                    idx_copy(bi, g + 2).start()

                @pl.when(g >= 2)
                def _drain_store():
                    store(bi, g - 2).wait()

                @pl.loop(0, half)
                def _pair_loop(p):
                    r0 = 2 * p
                    for j in range(2):
                        for c in range(dim // LANES):
                            stg[bi][p, pl.ds(j * dim + c * LANES, LANES)] = (
                                rows[bi][r0 + j, pl.ds(c * LANES, LANES)]
                                + pe_v[p, pl.ds(j * dim + c * LANES, LANES)]
                            )

                store(bi, g).start()

        store(n_chunks & 1, n_chunks - 2).wait()
        store(1 - (n_chunks & 1), n_chunks - 1).wait()

    return run(ids_flat, table, pe_pair)


def _tc_relayout(pairs, *, dim):
    n = 256
    row_ids = jnp.arange(2 * n)[:, None]
    col_ids = jnp.arange(n)[None, :]
    s_even = ((row_ids == 2 * col_ids) & (row_ids % 2 == 0)).astype(jnp.float32)
    s_odd = ((row_ids == 2 * col_ids + 1)).astype(jnp.float32)

    def body(x_ref, se_ref, so_ref, o_ref):
        x = x_ref[...]
        o_ref[...] = jax.lax.dot(
            se_ref[...], x[:, :dim], precision=jax.lax.Precision.HIGHEST,
        ) + jax.lax.dot(
            so_ref[...], x[:, dim:], precision=jax.lax.Precision.HIGHEST,
        )

    nblk = pairs.shape[0] // n
    return pl.pallas_call(
        body,
        grid=(nblk,),
        in_specs=[
            pl.BlockSpec((n, 2 * dim), lambda i: (i, 0)),
            pl.BlockSpec((2 * n, n), lambda i: (0, 0)),
            pl.BlockSpec((2 * n, n), lambda i: (0, 0)),
        ],
        out_specs=pl.BlockSpec((2 * n, dim), lambda i: (i, 0)),
        out_shape=jax.ShapeDtypeStruct((2 * pairs.shape[0], dim), jnp.float32),
    )(pairs, s_even, s_odd)


def kernel(input_ids, token_embedding):
    batch, seq_len = input_ids.shape
    dim = token_embedding.shape[1]
    ids_flat = input_ids.reshape(-1).astype(jnp.int32)
    pe = _make_sinusoidal_pe(seq_len, dim)
    reps = CHUNK // seq_len
    pe_pair = jnp.concatenate([pe] * reps, axis=0).reshape(CHUNK // 2, 2 * dim)
    pairs = _sc_embed(
        ids_flat, token_embedding, pe_pair,
        dim=dim, chunk=CHUNK, num_cores=NUM_CORES, num_subcores=NUM_SUBCORES,
    )
    out = _tc_relayout(pairs, dim=dim)
    return out.reshape(batch, seq_len, dim)


# speedup vs baseline: 2.1241x; 2.1241x over previous
import functools
import math

import jax
import jax.numpy as jnp
from jax import lax
from jax.experimental import pallas as pl
from jax.experimental.pallas import tpu as pltpu
from jax.experimental.pallas import tpu_sc as plsc

EMBED_DIM = 64
SEQ_LEN = 200
NUM_CORES = 2
NUM_SUBCORES = 16
LANES = 16
CHUNK = 400


def _make_sinusoidal_pe(seq_len, dim):
    position = jnp.arange(0, seq_len, dtype=jnp.float32)[:, None]
    div_term = jnp.exp(
        jnp.arange(0, dim, 2, dtype=jnp.float32) * -(math.log(10000.0) / dim)
    )
    pe = jnp.zeros((seq_len, dim), dtype=jnp.float32)
    pe = pe.at[:, 0::2].set(jnp.sin(position * div_term))
    pe = pe.at[:, 1::2].set(jnp.cos(position * div_term))
    return pe


def _sc_embed(ids_flat, table, pe_pair, *, dim, chunk, num_cores,
              num_subcores):
    num_workers = num_cores * num_subcores
    b = ids_flat.shape[0]
    b_per_w = b // num_workers
    n_chunks = b_per_w // chunk
    half = chunk // 2
    mesh = plsc.VectorSubcoreMesh(
        core_axis_name="c", subcore_axis_name="s",
        num_cores=num_cores, num_subcores=num_subcores,
    )

    @functools.partial(
        pl.kernel,
        out_type=jax.ShapeDtypeStruct((b // 2, 2 * dim), jnp.float32),
        mesh=mesh,
        scratch_types=[
            pltpu.VMEM((2, chunk), jnp.int32),
            pltpu.VMEM((chunk, dim), jnp.float32),
            pltpu.VMEM((chunk, dim), jnp.float32),
            pltpu.VMEM((half, 2 * dim), jnp.float32),
            pltpu.VMEM((half, 2 * dim), jnp.float32),
            pltpu.VMEM((half, 2 * dim), jnp.float32),
            pltpu.SemaphoreType.DMA,
            pltpu.SemaphoreType.DMA,
            pltpu.SemaphoreType.DMA,
            pltpu.SemaphoreType.DMA,
            pltpu.SemaphoreType.DMA,
            pltpu.SemaphoreType.DMA,
        ],
        compiler_params=pltpu.CompilerParams(use_tc_tiling_on_sc=False),
    )
    def run(ids_hbm, table_hbm, pe_hbm, out_hbm,
            idx_v, rows0_v, rows1_v, stg0_v, stg1_v, pe_v,
            isem0, isem1, gsem0, gsem1, osem0, osem1):
        rows = (rows0_v, rows1_v)
        stg = (stg0_v, stg1_v)
        isem = (isem0, isem1)
        gsem = (gsem0, gsem1)
        osem = (osem0, osem1)
        wid = lax.axis_index("s") * num_cores + lax.axis_index("c")
        base = wid * b_per_w
        pltpu.sync_copy(pe_hbm, pe_v)

        def idx_copy(bi, g):
            row0 = base + g * chunk
            return pltpu.make_async_copy(
                ids_hbm.at[pl.ds(row0, chunk)], idx_v.at[bi], isem[bi])

        def gather(bi, g):
            return pltpu.make_async_copy(
                table_hbm.at[idx_v.at[bi]], rows[bi], gsem[bi])

        def store(bi, g):
            p0 = (base + g * chunk) // 2
            return pltpu.make_async_copy(
                stg[bi], out_hbm.at[pl.ds(p0, half)], osem[bi])

        idx_copy(0, 0).start()
        idx_copy(1, 1).start()
        idx_copy(0, 0).wait()
        gather(0, 0).start()

        @pl.loop(0, n_chunks, step=2)
        def _chunk_loop(g0):
            for bi in range(2):
                g = g0 + bi
                oth = 1 - bi

                @pl.when(g + 1 < n_chunks)
                def _launch_next_gather():
                    idx_copy(oth, g + 1).wait()
                    gather(oth, g + 1).start()

                gather(bi, g).wait()

                @pl.when(g + 2 < n_chunks)
                def _prefetch_idx():
                    idx_copy(bi, g + 2).start()

                @pl.when(g >= 2)
                def _drain_store():
                    store(bi, g - 2).wait()

                @pl.loop(0, half)
                def _pair_loop(p):
                    r0 = 2 * p
                    for j in range(2):
                        for c in range(dim // LANES):
                            stg[bi][p, pl.ds(j * dim + c * LANES, LANES)] = (
                                rows[bi][r0 + j, pl.ds(c * LANES, LANES)]
                                + pe_v[p, pl.ds(j * dim + c * LANES, LANES)]
                            )

                store(bi, g).start()

        store(n_chunks & 1, n_chunks - 2).wait()
        store(1 - (n_chunks & 1), n_chunks - 1).wait()

    return run(ids_flat, table, pe_pair)


def _tc_relayout(pairs, *, dim):
    n = 256
    row_ids = jnp.arange(2 * n)[:, None]
    col_ids = jnp.arange(n)[None, :]
    s_even = ((row_ids == 2 * col_ids) & (row_ids % 2 == 0)).astype(jnp.float32)
    s_odd = ((row_ids == 2 * col_ids + 1)).astype(jnp.float32)

    def body(x_ref, se_ref, so_ref, o_ref):
        x = x_ref[...]
        o_ref[...] = jax.lax.dot(
            se_ref[...], x[:, :dim], precision=jax.lax.Precision.HIGHEST,
        ) + jax.lax.dot(
            so_ref[...], x[:, dim:], precision=jax.lax.Precision.HIGHEST,
        )

    nblk = pairs.shape[0] // n
    return pl.pallas_call(
        body,
        grid=(nblk,),
        in_specs=[
            pl.BlockSpec((n, 2 * dim), lambda i: (i, 0)),
            pl.BlockSpec((2 * n, n), lambda i: (0, 0)),
            pl.BlockSpec((2 * n, n), lambda i: (0, 0)),
        ],
        out_specs=pl.BlockSpec((2 * n, dim), lambda i: (i, 0)),
        out_shape=jax.ShapeDtypeStruct((2 * pairs.shape[0], dim), jnp.float32),
    )(pairs, s_even, s_odd)


def kernel(input_ids, token_embedding):
    batch, seq_len = input_ids.shape
    dim = token_embedding.shape[1]
    ids_flat = input_ids.reshape(-1).astype(jnp.int32)
    pe = _make_sinusoidal_pe(seq_len, dim)
    reps = CHUNK // seq_len
    pe_pair = jnp.concatenate([pe] * reps, axis=0).reshape(CHUNK // 2, 2 * dim)
    pairs = _sc_embed(
        ids_flat, token_embedding, pe_pair,
        dim=dim, chunk=CHUNK, num_cores=NUM_CORES, num_subcores=NUM_SUBCORES,
    )
    return pairs.reshape(batch, seq_len, dim)
